# baseline (speedup 1.0000x reference)
import jax
import jax.numpy as jnp
from jax.experimental import pallas as pl
from jax.experimental.pallas import tpu as pltpu

FEAT = 256
NTOT = 6768
B = 4096
BB = 512
NB = B // BB
FR = BB * NTOT // 128


def _body(x_ref, wt_ref, out_ref, loss_ref):
    i = pl.program_id(0)
    x = x_ref[...]
    z = jax.lax.dot_general(
        x.astype(jnp.bfloat16), wt_ref[...],
        (((1,), (0,)), ((), ())),
        preferred_element_type=jnp.float32,
    )
    zz = z[:, :128]
    out_ref[...] = jnp.broadcast_to(zz[:1, :], (FR, 128))

    @pl.when(i == 0)
    def _():
        loss_ref[0, 0] = 0.0


def kernel(inputs, targets, lut, queue):
    wt = jnp.concatenate([lut, queue], axis=0).T.astype(jnp.bfloat16)
    flat, loss = pl.pallas_call(
        _body,
        grid=(NB,),
        in_specs=[
            pl.BlockSpec((BB, FEAT), lambda i: (i, 0)),
            pl.BlockSpec((FEAT, NTOT), lambda i: (0, 0)),
        ],
        out_specs=[
            pl.BlockSpec((FR, 128), lambda i: (i, 0)),
            pl.BlockSpec(memory_space=pltpu.SMEM),
        ],
        out_shape=[
            jax.ShapeDtypeStruct((NB * FR, 128), jnp.float32),
            jax.ShapeDtypeStruct((1, 1), jnp.float32),
        ],
    )(inputs, wt)
    return (loss[0, 0], flat.reshape(B, NTOT))


# speedup vs baseline: 1.4683x; 1.4683x over previous
import functools

import jax
import jax.numpy as jnp
from jax import lax
from jax.experimental import pallas as pl
from jax.experimental.pallas import tpu as pltpu
from jax.experimental.pallas import tpu_sc as plsc

FEAT = 256
NCLS = 4768
NQ = 2000
NTOT = NCLS + NQ
SCALE = 30.0
B = 4096
BB = 512
NB = B // BB

NWORK = 32
BPW = B // NWORK
NLANE = 16


def _sc_dot_body(x_hbm, t_hbm, lut_hbm, dpart_hbm, idx_v, xrows, lrows, dp_v, sem):
    wid = lax.axis_index("s") * 2 + lax.axis_index("c")
    base = wid * BPW
    pltpu.sync_copy(t_hbm.at[pl.ds(base, BPW)], idx_v)
    pltpu.sync_copy(x_hbm.at[pl.ds(base, BPW)], xrows)
    pltpu.async_copy(lut_hbm.at[idx_v], lrows, sem).wait()

    def row16(rr, _):
        base_r = rr * 16
        for r2 in range(16):
            r = base_r + r2
            acc = jnp.zeros((NLANE,), jnp.float32)
            for j in range(FEAT // NLANE):
                xa = xrows[r, pl.ds(j * NLANE, NLANE)]
                la = lrows[r, pl.ds(j * NLANE, NLANE)]
                acc = acc + xa * la
            dp_v[r, :] = acc
        return _

    lax.fori_loop(0, BPW // 16, row16, 0)
    pltpu.sync_copy(dp_v, dpart_hbm.at[pl.ds(base, BPW)])


def _main_body(x_ref, wt_ref, dp_ref, logits_ref, loss_ref):
    i = pl.program_id(0)
    x = x_ref[...]
    nrm = jnp.sqrt(jnp.sum(x * x, axis=1, keepdims=True)) + 1e-12
    xn = x / nrm
    z = lax.dot_general(
        xn.astype(jnp.bfloat16), wt_ref[...],
        (((1,), (0,)), ((), ())),
        preferred_element_type=jnp.float32,
    ) * SCALE
    logits_ref[...] = z
    sumexp = jnp.sum(jnp.exp(z - SCALE), axis=1)
    tlogit = SCALE * jnp.sum(dp_ref[...], axis=1) / nrm[:, 0]
    partial = jnp.sum(SCALE + jnp.log(sumexp) - tlogit) * (1.0 / B)

    @pl.when(i == 0)
    def _():
        loss_ref[0, 0] = 0.0

    loss_ref[0, 0] += partial


def kernel(inputs, targets, lut, queue):
    wt = jnp.concatenate([lut, queue], axis=0).T.astype(jnp.bfloat16)

    mesh = plsc.VectorSubcoreMesh(core_axis_name="c", subcore_axis_name="s")
    dpart = functools.partial(
        pl.kernel,
        mesh=mesh,
        out_type=jax.ShapeDtypeStruct((B, NLANE), jnp.float32),
        scratch_types=[
            pltpu.VMEM((BPW,), jnp.int32),
            pltpu.VMEM((BPW, FEAT), jnp.float32),
            pltpu.VMEM((BPW, FEAT), jnp.float32),
            pltpu.VMEM((BPW, NLANE), jnp.float32),
            pltpu.SemaphoreType.DMA,
        ],
    )(_sc_dot_body)(inputs, targets, lut)

    logits, loss = pl.pallas_call(
        _main_body,
        grid=(NB,),
        in_specs=[
            pl.BlockSpec((BB, FEAT), lambda i: (i, 0)),
            pl.BlockSpec((FEAT, NTOT), lambda i: (0, 0)),
            pl.BlockSpec((BB, NLANE), lambda i: (i, 0)),
        ],
        out_specs=[
            pl.BlockSpec((BB, NTOT), lambda i: (i, 0)),
            pl.BlockSpec(memory_space=pltpu.SMEM),
        ],
        out_shape=[
            jax.ShapeDtypeStruct((B, NTOT), jnp.float32),
            jax.ShapeDtypeStruct((1, 1), jnp.float32),
        ],
    )(inputs, wt, dpart)
    return (loss[0, 0], logits)
